# baseline (speedup 1.0000x reference)
import jax
import jax.numpy as jnp
from jax.experimental import pallas as pl

_B, _NP, _NG, _T = 256, 128, 128, 10
_BB = 8


def _dotf(a, b):
    return jax.lax.dot_general(a, b, (((1,), (0,)), ((), ())),
                               precision=jax.lax.Precision.HIGHEST,
                               preferred_element_type=jnp.float32)


def _dm_kernel(gt_ref, gtt_ref, ini_ref, pred_ref, sini_ref, mask_ref,
               out_ref):
    t1 = 0.0
    t2 = 0.0
    t3 = 0.0
    for b in range(_BB):
        gt = gt_ref[b]
        gtr = jnp.concatenate([gt[_NG - 1:_NG], gt[:_NG - 1]], axis=0)
        u = gt - gtr
        ini2 = ini_ref[b]
        c2 = jnp.sum(u * u, axis=1, keepdims=True)
        w = jnp.sum(gtr * u, axis=1, keepdims=True)
        gn = jnp.sum(gtr * gtr, axis=1, keepdims=True)
        frc = jnp.maximum(-10.0 / c2, -2e21)

        c2_b = jnp.broadcast_to(c2, (_NG, _NP))
        frc_b = jnp.broadcast_to(frc, (_NG, _NP))

        ones_row = jnp.ones((1, _NP), jnp.float32)
        lhs_a = jnp.concatenate(
            [jnp.concatenate([u, w], axis=1),
             jnp.concatenate([2.0 * gtr, gn], axis=1)], axis=0)
        rhs_a = jnp.concatenate([-ini2, ones_row], axis=0)
        ma = _dotf(lhs_a, rhs_a)
        c1h = ma[:_NG]
        e2p = ma[_NG:]
        in_row = (ini2[0:1, :] * ini2[0:1, :]
                  + ini2[1:2, :] * ini2[1:2, :])

        sc = jnp.clip(c1h * frc_b + 0.5, 0.0, 9.0)
        k = sc.astype(jnp.int32).astype(jnp.float32) * 0.1
        m = jnp.maximum((e2p + in_row) + k * (2.0 * c1h + k * c2_b), 0.0)
        giota = jax.lax.broadcasted_iota(jnp.int32, (_NG, _NP), 0)
        mb = jax.lax.bitcast_convert_type(m, jnp.int32)
        pk = (mb & ~127) | giota
        pkmin = jnp.min(pk, axis=0, keepdims=True)
        sel = giota == (pkmin & 127)
        oh = sel.astype(jnp.float32)
        ohk = jnp.where(sel, k, 0.0)
        gtt = gtt_ref[b]
        gtr_r = jnp.concatenate([gtt[:, _NG - 1:_NG], gtt[:, :_NG - 1]],
                                axis=1)
        u_r = gtt - gtr_r
        n1 = _dotf(jnp.concatenate([gtr_r, u_r], axis=1),
                   jnp.concatenate([oh, ohk], axis=0))
        pxp = pred_ref[b, 0:1, :]
        pyp = pred_ref[b, 1:2, :]
        t1 = t1 + jnp.sum(jnp.abs(pxp - n1[0:1, :])
                          + jnp.abs(pyp - n1[1:2, :]))

        sini = sini_ref[b]
        in2 = jnp.sum(sini * sini, axis=1, keepdims=True)
        gxl = gtt[0:1, :]
        gyl = gtt[1:2, :]
        gn_row = gxl * gxl + gyl * gyl
        lhs_b = jnp.concatenate([-2.0 * sini, in2], axis=1)
        rhs_b = jnp.concatenate([gtt, jnp.ones((1, _NG), jnp.float32)],
                                axis=0)
        g2p = _dotf(lhs_b, rhs_b)
        d2 = jnp.maximum(g2p + gn_row, 0.0)
        piota = jax.lax.broadcasted_iota(jnp.int32, (_NP, _NG), 0)
        d2b = jax.lax.bitcast_convert_type(d2, jnp.int32)
        pk2 = (d2b & ~127) | piota
        pk2min = jnp.min(pk2, axis=0, keepdims=True)
        oh2 = (piota == (pk2min & 127)).astype(jnp.float32)
        n2 = _dotf(pred_ref[b], oh2)
        msk = mask_ref[b]
        t2 = t2 + jnp.sum((jnp.abs(n2[0:1, :] - gxl)
                           + jnp.abs(n2[1:2, :] - gyl)) * msk)
        t3 = t3 + jnp.sum(msk)

    lane = jax.lax.broadcasted_iota(jnp.int32, (1, 128), 1)
    vec = (jnp.where(lane == 0, t1, 0.0)
           + jnp.where(lane == 1, t2, 0.0)
           + jnp.where(lane == 2, t3, 0.0))

    @pl.when(pl.program_id(0) == 0)
    def _():
        out_ref[...] = jnp.zeros_like(out_ref)

    out_ref[...] += vec


@jax.jit
def kernel(ini_pred_poly, pred_polys_, gt_polys, keyPointsMask):
    ini_t = jnp.transpose(ini_pred_poly, (0, 2, 1))
    pred_t = jnp.transpose(pred_polys_, (0, 2, 1))
    gt_t = jnp.transpose(gt_polys, (0, 2, 1))
    mask3 = keyPointsMask[:, None, :]
    sums = pl.pallas_call(
        _dm_kernel,
        grid=(_B // _BB,),
        in_specs=[
            pl.BlockSpec((_BB, _NG, 2), lambda i: (i, 0, 0)),
            pl.BlockSpec((_BB, 2, _NG), lambda i: (i, 0, 0)),
            pl.BlockSpec((_BB, 2, _NP), lambda i: (i, 0, 0)),
            pl.BlockSpec((_BB, 2, _NP), lambda i: (i, 0, 0)),
            pl.BlockSpec((_BB, _NP, 2), lambda i: (i, 0, 0)),
            pl.BlockSpec((_BB, 1, _NG), lambda i: (i, 0, 0)),
        ],
        out_specs=pl.BlockSpec((1, 128), lambda i: (0, 0)),
        out_shape=jax.ShapeDtypeStruct((1, 128), jnp.float32),
    )(gt_polys, gt_t, ini_t, pred_t, ini_pred_poly, mask3)
    t1 = sums[0, 0]
    t2 = sums[0, 1]
    t3 = sums[0, 2]
    loss1 = t1 / (_B * _NP * 2)
    loss = t2 / (2.0 * t3 + 1.0) + loss1
    return loss / 2.0


# speedup vs baseline: 1.2436x; 1.2436x over previous
import jax
import jax.numpy as jnp
from jax.experimental import pallas as pl

_B, _NP, _NG, _T = 256, 128, 128, 10
_BB = 16


def _dotf(a, b):
    return jax.lax.dot_general(a, b, (((1,), (0,)), ((), ())),
                               precision=jax.lax.Precision.HIGHEST,
                               preferred_element_type=jnp.float32)


def _dm_kernel(gt_ref, gtt_ref, ini_ref, pred_ref, sini_ref, mask_ref,
               out_ref):
    t1 = 0.0
    t2 = 0.0
    t3 = 0.0
    for b in range(_BB):
        gt = gt_ref[b]
        gtr = jnp.concatenate([gt[_NG - 1:_NG], gt[:_NG - 1]], axis=0)
        u = gt - gtr
        ini2 = ini_ref[b]
        c2 = jnp.sum(u * u, axis=1, keepdims=True)
        w = jnp.sum(gtr * u, axis=1, keepdims=True)
        gn = jnp.sum(gtr * gtr, axis=1, keepdims=True)
        frc = jnp.maximum(-10.0 / c2, -2e21)

        c2_b = jnp.broadcast_to(c2, (_NG, _NP))
        w_b = jnp.broadcast_to(w, (_NG, _NP))
        gn_b = jnp.broadcast_to(gn, (_NG, _NP))
        frc_b = jnp.broadcast_to(frc, (_NG, _NP))

        uc = _dotf(u, ini2)
        gc = _dotf(gtr, ini2)
        in_row = (ini2[0:1, :] * ini2[0:1, :]
                  + ini2[1:2, :] * ini2[1:2, :])

        c1h = w_b - uc
        sc = jnp.clip(c1h * frc_b + 0.5, 0.0, 9.0)
        k = sc.astype(jnp.int32).astype(jnp.float32) * 0.1
        e2 = (gn_b - 2.0 * gc) + in_row
        m = jnp.maximum(e2 + k * (2.0 * c1h + k * c2_b), 0.0)
        giota = jax.lax.broadcasted_iota(jnp.int32, (_NG, _NP), 0)
        mb = jax.lax.bitcast_convert_type(m, jnp.int32)
        pk = (mb & ~127) | giota
        pkmin = jnp.min(pk, axis=0, keepdims=True)
        sel = giota == (pkmin & 127)
        oh = sel.astype(jnp.float32)
        ohk = jnp.where(sel, k, 0.0)
        gtt = gtt_ref[b]
        gtr_r = jnp.concatenate([gtt[:, _NG - 1:_NG], gtt[:, :_NG - 1]],
                                axis=1)
        u_r = gtt - gtr_r
        n1 = _dotf(gtr_r, oh) + _dotf(u_r, ohk)
        pxp = pred_ref[b, 0:1, :]
        pyp = pred_ref[b, 1:2, :]
        t1 = t1 + jnp.sum(jnp.abs(pxp - n1[0:1, :])
                          + jnp.abs(pyp - n1[1:2, :]))

        sini = sini_ref[b]
        in2 = jnp.sum(sini * sini, axis=1, keepdims=True)
        in2_b = jnp.broadcast_to(in2, (_NP, _NG))
        gxl = gtt[0:1, :]
        gyl = gtt[1:2, :]
        gn_row = gxl * gxl + gyl * gyl
        g2 = _dotf(sini, gtt)
        d2 = jnp.maximum((in2_b - 2.0 * g2) + gn_row, 0.0)
        piota = jax.lax.broadcasted_iota(jnp.int32, (_NP, _NG), 0)
        d2b = jax.lax.bitcast_convert_type(d2, jnp.int32)
        pk2 = (d2b & ~127) | piota
        pk2min = jnp.min(pk2, axis=0, keepdims=True)
        oh2 = (piota == (pk2min & 127)).astype(jnp.float32)
        n2 = _dotf(pred_ref[b], oh2)
        msk = mask_ref[b]
        t2 = t2 + jnp.sum((jnp.abs(n2[0:1, :] - gxl)
                           + jnp.abs(n2[1:2, :] - gyl)) * msk)
        t3 = t3 + jnp.sum(msk)

    lane = jax.lax.broadcasted_iota(jnp.int32, (1, 128), 1)
    vec = (jnp.where(lane == 0, t1, 0.0)
           + jnp.where(lane == 1, t2, 0.0)
           + jnp.where(lane == 2, t3, 0.0))

    @pl.when(pl.program_id(0) == 0)
    def _():
        out_ref[...] = jnp.zeros_like(out_ref)

    out_ref[...] += vec


@jax.jit
def kernel(ini_pred_poly, pred_polys_, gt_polys, keyPointsMask):
    ini_t = jnp.transpose(ini_pred_poly, (0, 2, 1))
    pred_t = jnp.transpose(pred_polys_, (0, 2, 1))
    gt_t = jnp.transpose(gt_polys, (0, 2, 1))
    mask3 = keyPointsMask[:, None, :]
    sums = pl.pallas_call(
        _dm_kernel,
        grid=(_B // _BB,),
        in_specs=[
            pl.BlockSpec((_BB, _NG, 2), lambda i: (i, 0, 0)),
            pl.BlockSpec((_BB, 2, _NG), lambda i: (i, 0, 0)),
            pl.BlockSpec((_BB, 2, _NP), lambda i: (i, 0, 0)),
            pl.BlockSpec((_BB, 2, _NP), lambda i: (i, 0, 0)),
            pl.BlockSpec((_BB, _NP, 2), lambda i: (i, 0, 0)),
            pl.BlockSpec((_BB, 1, _NG), lambda i: (i, 0, 0)),
        ],
        out_specs=pl.BlockSpec((1, 128), lambda i: (0, 0)),
        out_shape=jax.ShapeDtypeStruct((1, 128), jnp.float32),
    )(gt_polys, gt_t, ini_t, pred_t, ini_pred_poly, mask3)
    t1 = sums[0, 0]
    t2 = sums[0, 1]
    t3 = sums[0, 2]
    loss1 = t1 / (_B * _NP * 2)
    loss = t2 / (2.0 * t3 + 1.0) + loss1
    return loss / 2.0


# speedup vs baseline: 1.2713x; 1.0223x over previous
import jax
import jax.numpy as jnp
from jax.experimental import pallas as pl

_B, _NP, _NG, _T = 256, 128, 128, 10
_BB = 32


def _dotf(a, b):
    return jax.lax.dot_general(a, b, (((1,), (0,)), ((), ())),
                               precision=jax.lax.Precision.HIGHEST,
                               preferred_element_type=jnp.float32)


def _dm_kernel(gt_ref, gtt_ref, ini_ref, pred_ref, sini_ref, mask_ref,
               out_ref):
    t1 = 0.0
    t2 = 0.0
    t3 = 0.0
    for b in range(_BB):
        gt = gt_ref[b]
        gtr = jnp.concatenate([gt[_NG - 1:_NG], gt[:_NG - 1]], axis=0)
        u = gt - gtr
        ini2 = ini_ref[b]
        c2 = jnp.sum(u * u, axis=1, keepdims=True)
        w = jnp.sum(gtr * u, axis=1, keepdims=True)
        gn = jnp.sum(gtr * gtr, axis=1, keepdims=True)
        frc = jnp.maximum(-10.0 / c2, -2e21)

        c2_b = jnp.broadcast_to(c2, (_NG, _NP))
        w_b = jnp.broadcast_to(w, (_NG, _NP))
        gn_b = jnp.broadcast_to(gn, (_NG, _NP))
        frc_b = jnp.broadcast_to(frc, (_NG, _NP))

        uc = _dotf(u, ini2)
        gc = _dotf(gtr, ini2)
        in_row = (ini2[0:1, :] * ini2[0:1, :]
                  + ini2[1:2, :] * ini2[1:2, :])

        c1h = w_b - uc
        sc = jnp.clip(c1h * frc_b + 0.5, 0.0, 9.0)
        k = sc.astype(jnp.int32).astype(jnp.float32) * 0.1
        e2 = (gn_b - 2.0 * gc) + in_row
        m = jnp.maximum(e2 + k * (2.0 * c1h + k * c2_b), 0.0)
        giota = jax.lax.broadcasted_iota(jnp.int32, (_NG, _NP), 0)
        mb = jax.lax.bitcast_convert_type(m, jnp.int32)
        pk = (mb & ~127) | giota
        pkmin = jnp.min(pk, axis=0, keepdims=True)
        sel = giota == (pkmin & 127)
        oh = sel.astype(jnp.float32)
        ohk = jnp.where(sel, k, 0.0)
        gtt = gtt_ref[b]
        gtr_r = jnp.concatenate([gtt[:, _NG - 1:_NG], gtt[:, :_NG - 1]],
                                axis=1)
        u_r = gtt - gtr_r
        n1 = _dotf(gtr_r, oh) + _dotf(u_r, ohk)
        pxp = pred_ref[b, 0:1, :]
        pyp = pred_ref[b, 1:2, :]
        t1 = t1 + jnp.sum(jnp.abs(pxp - n1[0:1, :])
                          + jnp.abs(pyp - n1[1:2, :]))

        sini = sini_ref[b]
        in2 = jnp.sum(sini * sini, axis=1, keepdims=True)
        in2_b = jnp.broadcast_to(in2, (_NP, _NG))
        gxl = gtt[0:1, :]
        gyl = gtt[1:2, :]
        gn_row = gxl * gxl + gyl * gyl
        g2 = _dotf(sini, gtt)
        d2 = jnp.maximum((in2_b - 2.0 * g2) + gn_row, 0.0)
        piota = jax.lax.broadcasted_iota(jnp.int32, (_NP, _NG), 0)
        d2b = jax.lax.bitcast_convert_type(d2, jnp.int32)
        pk2 = (d2b & ~127) | piota
        pk2min = jnp.min(pk2, axis=0, keepdims=True)
        oh2 = (piota == (pk2min & 127)).astype(jnp.float32)
        n2 = _dotf(pred_ref[b], oh2)
        msk = mask_ref[b]
        t2 = t2 + jnp.sum((jnp.abs(n2[0:1, :] - gxl)
                           + jnp.abs(n2[1:2, :] - gyl)) * msk)
        t3 = t3 + jnp.sum(msk)

    lane = jax.lax.broadcasted_iota(jnp.int32, (1, 128), 1)
    vec = (jnp.where(lane == 0, t1, 0.0)
           + jnp.where(lane == 1, t2, 0.0)
           + jnp.where(lane == 2, t3, 0.0))

    @pl.when(pl.program_id(0) == 0)
    def _():
        out_ref[...] = jnp.zeros_like(out_ref)

    out_ref[...] += vec


@jax.jit
def kernel(ini_pred_poly, pred_polys_, gt_polys, keyPointsMask):
    ini_t = jnp.transpose(ini_pred_poly, (0, 2, 1))
    pred_t = jnp.transpose(pred_polys_, (0, 2, 1))
    gt_t = jnp.transpose(gt_polys, (0, 2, 1))
    mask3 = keyPointsMask[:, None, :]
    sums = pl.pallas_call(
        _dm_kernel,
        grid=(_B // _BB,),
        in_specs=[
            pl.BlockSpec((_BB, _NG, 2), lambda i: (i, 0, 0)),
            pl.BlockSpec((_BB, 2, _NG), lambda i: (i, 0, 0)),
            pl.BlockSpec((_BB, 2, _NP), lambda i: (i, 0, 0)),
            pl.BlockSpec((_BB, 2, _NP), lambda i: (i, 0, 0)),
            pl.BlockSpec((_BB, _NP, 2), lambda i: (i, 0, 0)),
            pl.BlockSpec((_BB, 1, _NG), lambda i: (i, 0, 0)),
        ],
        out_specs=pl.BlockSpec((1, 128), lambda i: (0, 0)),
        out_shape=jax.ShapeDtypeStruct((1, 128), jnp.float32),
    )(gt_polys, gt_t, ini_t, pred_t, ini_pred_poly, mask3)
    t1 = sums[0, 0]
    t2 = sums[0, 1]
    t3 = sums[0, 2]
    loss1 = t1 / (_B * _NP * 2)
    loss = t2 / (2.0 * t3 + 1.0) + loss1
    return loss / 2.0
